# Initial kernel scaffold; baseline (speedup 1.0000x reference)
import jax
import jax.numpy as jnp
from jax.experimental import pallas as pl


def kernel(predictions, targets, group_labels):
    raise NotImplementedError("write your pallas kernel here")


# speedup vs baseline: 67.9373x; 67.9373x over previous
import functools

import jax
import jax.numpy as jnp
from jax import lax
from jax.experimental import pallas as pl
from jax.experimental.pallas import tpu as pltpu
from jax.experimental.pallas import tpu_sc as plsc

_N = 3200000
_G = 8
_NW = 32
_PER_W = _N // _NW
_CH = 10000
_NCH = _PER_W // _CH
_L = 16
_VECS = _CH // _L

_mesh = plsc.VectorSubcoreMesh(core_axis_name="c", subcore_axis_name="s")


@functools.partial(
    pl.kernel,
    mesh=_mesh,
    compiler_params=pltpu.CompilerParams(needs_layout_passes=False),
    out_type=[
        jax.ShapeDtypeStruct((_NW, 128), jnp.float32),
        jax.ShapeDtypeStruct((_NW, 128), jnp.float32),
    ],
    scratch_types=[
        pltpu.VMEM((_CH,), jnp.float32),
        pltpu.VMEM((_CH,), jnp.float32),
        pltpu.VMEM((_CH,), jnp.int32),
        pltpu.VMEM((128,), jnp.float32),
        pltpu.VMEM((128,), jnp.float32),
    ],
)
def _sc_partials(p_hbm, t_hbm, lab_hbm, sums_out, cnts_out,
                 pbuf, tbuf, lbuf, sacc, cacc):
    wid = lax.axis_index("s") * 2 + lax.axis_index("c")
    base = wid * _PER_W
    zeros = jnp.zeros((_L,), jnp.float32)
    ones = jnp.ones((_L,), jnp.float32)
    lane = lax.iota(jnp.int32, _L)
    for g in range(_G):
        sacc[pl.ds(g * _L, _L)] = zeros
        cacc[pl.ds(g * _L, _L)] = zeros

    def chunk_body(ci, carry):
        off = base + ci * _CH
        pltpu.sync_copy(p_hbm.at[pl.ds(off, _CH)], pbuf)
        pltpu.sync_copy(t_hbm.at[pl.ds(off, _CH)], tbuf)
        pltpu.sync_copy(lab_hbm.at[pl.ds(off, _CH)], lbuf)

        def vec_body(vi, c2):
            s = vi * _L
            p = pbuf[pl.ds(s, _L)]
            t = tbuf[pl.ds(s, _L)]
            l = lbuf[pl.ds(s, _L)]
            d = p - t
            idx = l * _L + lane
            plsc.addupdate_scatter(sacc, [idx], d * d)
            plsc.addupdate_scatter(cacc, [idx], ones)
            return c2

        return lax.fori_loop(0, _VECS, vec_body, carry)

    lax.fori_loop(0, _NCH, chunk_body, 0)
    pltpu.sync_copy(sacc, sums_out.at[wid])
    pltpu.sync_copy(cacc, cnts_out.at[wid])


def _finalize_body(s_ref, c_ref, o_ref):
    s = jnp.sum(s_ref[...], axis=0, keepdims=True)
    c = jnp.sum(c_ref[...], axis=0, keepdims=True)
    gid = lax.broadcasted_iota(jnp.int32, (1, 128), 1) // _L
    total = jnp.sum(s)
    base = total / _N
    mse_sum = jnp.float32(0.0)
    mses = []
    for g in range(_G):
        sg = jnp.sum(jnp.where(gid == g, s, 0.0))
        cg = jnp.sum(jnp.where(gid == g, c, 0.0))
        m = sg / jnp.maximum(cg, 1.0)
        mses.append(m)
        mse_sum = mse_sum + m
    mu = mse_sum / _G
    var = jnp.float32(0.0)
    for g in range(_G):
        dm = mses[g] - mu
        var = var + dm * dm
    var = var / (_G - 1)
    o_ref[...] = jnp.full((1, 1), base + var, jnp.float32)


_finalize = pl.pallas_call(
    _finalize_body,
    out_shape=jax.ShapeDtypeStruct((1, 1), jnp.float32),
)


def kernel(predictions, targets, group_labels):
    labels = group_labels.astype(jnp.int32)
    sums, cnts = _sc_partials(predictions, targets, labels)
    return _finalize(sums, cnts)[0, 0]


# speedup vs baseline: 89.3029x; 1.3145x over previous
import functools

import jax
import jax.numpy as jnp
from jax import lax
from jax.experimental import pallas as pl
from jax.experimental.pallas import tpu as pltpu
from jax.experimental.pallas import tpu_sc as plsc

_N = 3200000
_G = 8
_NW = 32
_PER_W = _N // _NW
_CH = 10000
_NCH = _PER_W // _CH
_L = 16
_VECS = _CH // _L

_mesh = plsc.VectorSubcoreMesh(core_axis_name="c", subcore_axis_name="s")


@functools.partial(
    pl.kernel,
    mesh=_mesh,
    compiler_params=pltpu.CompilerParams(needs_layout_passes=False),
    out_type=[
        jax.ShapeDtypeStruct((_NW, 128), jnp.float32),
        jax.ShapeDtypeStruct((_NW, 128), jnp.float32),
    ],
    scratch_types=[
        pltpu.VMEM((_CH,), jnp.float32),
        pltpu.VMEM((_CH,), jnp.float32),
        pltpu.VMEM((_CH,), jnp.int32),
        pltpu.VMEM((_CH,), jnp.float32),
        pltpu.VMEM((_CH,), jnp.float32),
        pltpu.VMEM((_CH,), jnp.int32),
        pltpu.VMEM((128,), jnp.float32),
        pltpu.VMEM((128,), jnp.float32),
        pltpu.SemaphoreType.DMA,
        pltpu.SemaphoreType.DMA,
    ],
)
def _sc_partials(p_hbm, t_hbm, lab_hbm, sums_out, cnts_out,
                 pbuf0, tbuf0, lbuf0, pbuf1, tbuf1, lbuf1,
                 sacc, cacc, sem0, sem1):
    wid = lax.axis_index("s") * 2 + lax.axis_index("c")
    base = wid * _PER_W
    zeros = jnp.zeros((_L,), jnp.float32)
    ones = jnp.ones((_L,), jnp.float32)
    lane = lax.iota(jnp.int32, _L)
    for g in range(_G):
        sacc[pl.ds(g * _L, _L)] = zeros
        cacc[pl.ds(g * _L, _L)] = zeros

    slots = ((pbuf0, tbuf0, lbuf0, sem0), (pbuf1, tbuf1, lbuf1, sem1))

    def start_chunk(ci):
        pb, tb, lb, sem = slots[ci % 2]
        off = base + ci * _CH
        return (pltpu.async_copy(p_hbm.at[pl.ds(off, _CH)], pb, sem),
                pltpu.async_copy(t_hbm.at[pl.ds(off, _CH)], tb, sem),
                pltpu.async_copy(lab_hbm.at[pl.ds(off, _CH)], lb, sem))

    _U = 5

    def compute_chunk(ci):
        pb, tb, lb, _ = slots[ci % 2]

        def vec_body(vi, carry):
            s0 = vi * (_L * _U)
            for k in range(_U):
                s = s0 + k * _L
                p = pb[pl.ds(s, _L)]
                t = tb[pl.ds(s, _L)]
                l = lb[pl.ds(s, _L)]
                d = p - t
                idx = l * _L + lane
                plsc.addupdate_scatter(sacc, [idx], d * d)
                plsc.addupdate_scatter(cacc, [idx], ones)
            return carry

        lax.fori_loop(0, _VECS // _U, vec_body, 0)

    handles = start_chunk(0)
    for ci in range(_NCH):
        nxt = start_chunk(ci + 1) if ci + 1 < _NCH else None
        for h in handles:
            h.wait()
        compute_chunk(ci)
        handles = nxt

    pltpu.sync_copy(sacc, sums_out.at[wid])
    pltpu.sync_copy(cacc, cnts_out.at[wid])


def _finalize_body(s_ref, c_ref, o_ref):
    s = jnp.sum(s_ref[...], axis=0, keepdims=True)
    c = jnp.sum(c_ref[...], axis=0, keepdims=True)
    gid = lax.broadcasted_iota(jnp.int32, (1, 128), 1) // _L
    total = jnp.sum(s)
    base = total / _N
    mse_sum = jnp.float32(0.0)
    mses = []
    for g in range(_G):
        sg = jnp.sum(jnp.where(gid == g, s, 0.0))
        cg = jnp.sum(jnp.where(gid == g, c, 0.0))
        m = sg / jnp.maximum(cg, 1.0)
        mses.append(m)
        mse_sum = mse_sum + m
    mu = mse_sum / _G
    var = jnp.float32(0.0)
    for g in range(_G):
        dm = mses[g] - mu
        var = var + dm * dm
    var = var / (_G - 1)
    o_ref[...] = jnp.full((1, 1), base + var, jnp.float32)


_finalize = pl.pallas_call(
    _finalize_body,
    out_shape=jax.ShapeDtypeStruct((1, 1), jnp.float32),
)


def kernel(predictions, targets, group_labels):
    labels = group_labels.astype(jnp.int32)
    sums, cnts = _sc_partials(predictions, targets, labels)
    return _finalize(sums, cnts)[0, 0]


# speedup vs baseline: 148.8911x; 1.6673x over previous
import functools

import jax
import jax.numpy as jnp
from jax import lax
from jax.experimental import pallas as pl
from jax.experimental.pallas import tpu as pltpu
from jax.experimental.pallas import tpu_sc as plsc

_N = 3200000
_G = 8
_NW = 32
_PER_W = _N // _NW
_CH = 10000
_NCH = _PER_W // _CH
_L = 16
_VECS = _CH // _L

_mesh = plsc.VectorSubcoreMesh(core_axis_name="c", subcore_axis_name="s")


@functools.partial(
    pl.kernel,
    mesh=_mesh,
    compiler_params=pltpu.CompilerParams(needs_layout_passes=False),
    out_type=[
        jax.ShapeDtypeStruct((_NW, 128), jnp.float32),
        jax.ShapeDtypeStruct((_NW, 128), jnp.float32),
    ],
    scratch_types=[
        pltpu.VMEM((_CH,), jnp.float32),
        pltpu.VMEM((_CH,), jnp.float32),
        pltpu.VMEM((_CH,), jnp.int32),
        pltpu.VMEM((_CH,), jnp.float32),
        pltpu.VMEM((_CH,), jnp.float32),
        pltpu.VMEM((_CH,), jnp.int32),
        pltpu.VMEM((128,), jnp.float32),
        pltpu.VMEM((128,), jnp.float32),
        pltpu.SemaphoreType.DMA,
        pltpu.SemaphoreType.DMA,
    ],
)
def _sc_partials(p_hbm, t_hbm, lab_hbm, sums_out, cnts_out,
                 pbuf0, tbuf0, lbuf0, pbuf1, tbuf1, lbuf1,
                 sacc, cacc, sem0, sem1):
    wid = lax.axis_index("s") * 2 + lax.axis_index("c")
    base = wid * _PER_W
    zeros = jnp.zeros((_L,), jnp.float32)
    ones = jnp.ones((_L,), jnp.float32)
    lane = lax.iota(jnp.int32, _L)
    for g in range(_G):
        sacc[pl.ds(g * _L, _L)] = zeros
        cacc[pl.ds(g * _L, _L)] = zeros

    slots = ((pbuf0, tbuf0, lbuf0, sem0), (pbuf1, tbuf1, lbuf1, sem1))

    def start_chunk(ci):
        pb, tb, lb, sem = slots[ci % 2]
        off = base + ci * _CH
        return (pltpu.async_copy(p_hbm.at[pl.ds(off, _CH)], pb, sem),
                pltpu.async_copy(t_hbm.at[pl.ds(off, _CH)], tb, sem),
                pltpu.async_copy(lab_hbm.at[pl.ds(off, _CH)], lb, sem))

    _U = 5

    def compute_chunk(ci):
        pb, tb, lb, _ = slots[ci % 2]

        def vec_body(vi, carry):
            s0 = vi * (_L * _U)
            ps, ts, ls = [], [], []
            for k in range(_U):
                s = s0 + k * _L
                ps.append(pb[pl.ds(s, _L)])
                ts.append(tb[pl.ds(s, _L)])
                ls.append(lb[pl.ds(s, _L)])
            sqs, idxs = [], []
            for k in range(_U):
                d = ps[k] - ts[k]
                sqs.append(d * d)
                idxs.append(ls[k] * _L + lane)
            for k in range(_U):
                plsc.addupdate_scatter(sacc, [idxs[k]], sqs[k])
                plsc.addupdate_scatter(cacc, [idxs[k]], ones)
            return carry

        lax.fori_loop(0, _VECS // _U, vec_body, 0)

    handles = start_chunk(0)
    for ci in range(_NCH):
        nxt = start_chunk(ci + 1) if ci + 1 < _NCH else None
        for h in handles:
            h.wait()
        compute_chunk(ci)
        handles = nxt

    pltpu.sync_copy(sacc, sums_out.at[wid])
    pltpu.sync_copy(cacc, cnts_out.at[wid])


def _finalize_body(s_ref, c_ref, o_ref):
    s = jnp.sum(s_ref[...], axis=0, keepdims=True)
    c = jnp.sum(c_ref[...], axis=0, keepdims=True)
    gid = lax.broadcasted_iota(jnp.int32, (1, 128), 1) // _L
    total = jnp.sum(s)
    base = total / _N
    mse_sum = jnp.float32(0.0)
    mses = []
    for g in range(_G):
        sg = jnp.sum(jnp.where(gid == g, s, 0.0))
        cg = jnp.sum(jnp.where(gid == g, c, 0.0))
        m = sg / jnp.maximum(cg, 1.0)
        mses.append(m)
        mse_sum = mse_sum + m
    mu = mse_sum / _G
    var = jnp.float32(0.0)
    for g in range(_G):
        dm = mses[g] - mu
        var = var + dm * dm
    var = var / (_G - 1)
    o_ref[...] = jnp.full((1, 1), base + var, jnp.float32)


_finalize = pl.pallas_call(
    _finalize_body,
    out_shape=jax.ShapeDtypeStruct((1, 1), jnp.float32),
)


def kernel(predictions, targets, group_labels):
    labels = group_labels.astype(jnp.int32)
    sums, cnts = _sc_partials(predictions, targets, labels)
    return _finalize(sums, cnts)[0, 0]
